# baseline (speedup 1.0000x reference)
import functools

import jax
import jax.numpy as jnp
from jax import lax
from jax.experimental import pallas as pl
from jax.experimental.pallas import tpu as pltpu
from jax.experimental.pallas import tpu_sc as plsc

N = 10000
E = 320000
D = 128
H = 128
C = 40

NC = 2
NS = 16
NW = NC * NS
EPW = E // NW
CHUNK = 125
NCHUNK = EPW // CHUNK
HALF = NCHUNK // 2
NB = 2
ROWS_PS = 640
NPAD = NS * ROWS_PS
DEGW = 128

_mesh = plsc.VectorSubcoreMesh(core_axis_name="c", subcore_axis_name="s")


def _seg_sum_body(h_hbm, src_hbm, dst_hbm, zf_hbm, agg_hbm, agg_sh,
                  src_idx, dst_idx, rows, *sems):
    c = lax.axis_index("c")
    s = lax.axis_index("s")
    wid = c * NS + s
    gsem = sems[:NB]
    ssem = sems[NB:]

    r0 = s * ROWS_PS
    pltpu.sync_copy(zf_hbm, agg_sh.at[pl.ds(r0, ROWS_PS)])

    def g_issue(i, b):
        pltpu.async_copy(h_hbm.at[src_idx.at[i]], rows.at[b], gsem[b])

    def g_wait(i, b):
        pltpu.make_async_copy(h_hbm.at[src_idx.at[i]], rows.at[b],
                              gsem[b]).wait()

    def s_issue(i, b):
        pltpu.async_copy(rows.at[b], agg_sh.at[dst_idx.at[i]], ssem[b],
                         add=True)

    def s_wait(i, b):
        pltpu.make_async_copy(rows.at[b], agg_sh.at[dst_idx.at[i]],
                              ssem[b]).wait()

    plsc.subcore_barrier()

    for h in range(2):
        pltpu.sync_copy(src_hbm.at[wid, pl.ds(h * HALF, HALF)], src_idx)
        pltpu.sync_copy(dst_hbm.at[wid, pl.ds(h * HALF, HALF)], dst_idx)

        for b in range(NB):
            g_issue(b, b)

        @pl.loop(0, HALF // NB - 1)
        def _(j):
            i0 = j * NB
            for b in range(NB):
                g_wait(i0 + b, b)
                s_issue(i0 + b, b)
            for b in range(NB):
                s_wait(i0 + b, b)
                g_issue(i0 + NB + b, b)

        i0 = HALF - NB
        for b in range(NB):
            g_wait(i0 + b, b)
            s_issue(i0 + b, b)
        for b in range(NB):
            s_wait(i0 + b, b)

    plsc.subcore_barrier()
    pltpu.sync_copy(agg_sh.at[pl.ds(r0, ROWS_PS)],
                    agg_hbm.at[c, pl.ds(r0, ROWS_PS)])


def _deg_body(dst_hbm, zd_hbm, ones_hbm, deg_hbm, deg_sh, dst_idx, ones_v,
              *sems):
    c = lax.axis_index("c")
    s = lax.axis_index("s")
    wid = c * NS + s
    r0 = s * ROWS_PS
    pltpu.sync_copy(zd_hbm, deg_sh.at[pl.ds(r0, ROWS_PS)])
    pltpu.sync_copy(ones_hbm, ones_v)
    pltpu.sync_copy(dst_hbm.at[wid], dst_idx)
    plsc.subcore_barrier()

    def s_issue(i, b):
        pltpu.async_copy(ones_v, deg_sh.at[dst_idx.at[i]], sems[b], add=True)

    def s_wait(i, b):
        pltpu.make_async_copy(ones_v, deg_sh.at[dst_idx.at[i]],
                              sems[b]).wait()

    for b in range(NB):
        s_issue(b, b)

    @pl.loop(0, NCHUNK // NB - 1)
    def _(j):
        i0 = j * NB
        for b in range(NB):
            s_wait(i0 + b, b)
            s_issue(i0 + NB + b, b)

    i0 = NCHUNK - NB
    for b in range(NB):
        s_wait(i0 + b, b)

    plsc.subcore_barrier()
    pltpu.sync_copy(deg_sh.at[pl.ds(r0, ROWS_PS)],
                    deg_hbm.at[c, pl.ds(r0, ROWS_PS)])


_seg_sum = pl.kernel(
    _seg_sum_body,
    out_type=jax.ShapeDtypeStruct((NC, NPAD, D), jnp.float32),
    mesh=_mesh,
    scratch_types=[
        pltpu.VMEM_SHARED((NPAD, D), jnp.float32),
        pltpu.VMEM((HALF, CHUNK), jnp.int32),
        pltpu.VMEM((HALF, CHUNK), jnp.int32),
        pltpu.VMEM((NB, CHUNK, D), jnp.float32),
    ] + [pltpu.SemaphoreType.DMA] * (2 * NB),
)

_deg_count = pl.kernel(
    _deg_body,
    out_type=jax.ShapeDtypeStruct((NC, NPAD, DEGW), jnp.float32),
    mesh=_mesh,
    scratch_types=[
        pltpu.VMEM_SHARED((NPAD, DEGW), jnp.float32),
        pltpu.VMEM((NCHUNK, CHUNK), jnp.int32),
        pltpu.VMEM((CHUNK, DEGW), jnp.float32),
    ] + [pltpu.SemaphoreType.DMA] * NB,
)


BN = 1000


def _layer_body(final, h_ref, aggp_ref, degp_ref, ws_ref, wn_ref, b_ref,
                wfc_ref, bfc_ref, out_ref):
    agg = aggp_ref[0] + aggp_ref[1]
    deg = degp_ref[0, :, 0] + degp_ref[1, :, 0]
    recip = 1.0 / jnp.maximum(deg, 1.0)
    neigh = agg * recip[:, None]
    hh = (jnp.dot(h_ref[...], ws_ref[...], preferred_element_type=jnp.float32)
          + jnp.dot(neigh, wn_ref[...], preferred_element_type=jnp.float32)
          + b_ref[...])
    hh = jnp.maximum(hh, 0.0)
    if final:
        out_ref[...] = (jnp.dot(hh, wfc_ref[...],
                                preferred_element_type=jnp.float32)
                        + bfc_ref[...])
    else:
        out_ref[...] = hh


def _make_layer(final):
    specs = [
        pl.BlockSpec((BN, D), lambda i: (i, 0)),
        pl.BlockSpec((NC, BN, D), lambda i: (0, i, 0)),
        pl.BlockSpec((NC, BN, DEGW), lambda i: (0, i, 0)),
        pl.BlockSpec((D, H), lambda i: (0, 0)),
        pl.BlockSpec((D, H), lambda i: (0, 0)),
        pl.BlockSpec((1, H), lambda i: (0, 0)),
        pl.BlockSpec((H, C), lambda i: (0, 0)),
        pl.BlockSpec((1, C), lambda i: (0, 0)),
    ]
    width = C if final else H
    return pl.pallas_call(
        functools.partial(_layer_body, final),
        grid=(N // BN,),
        in_specs=specs,
        out_specs=pl.BlockSpec((BN, width), lambda i: (i, 0)),
        out_shape=jax.ShapeDtypeStruct((N, width), jnp.float32),
    )


_layer_hidden = _make_layer(False)
_layer_final = _make_layer(True)


def kernel(x, edge_index, W_self0, W_neigh0, b0, W_self1, W_neigh1, b1,
           W_fc, b_fc):
    src = edge_index[0].reshape(NW, NCHUNK, CHUNK)
    dst = edge_index[1].reshape(NW, NCHUNK, CHUNK)
    zf = jnp.zeros((ROWS_PS, D), jnp.float32)
    zd = jnp.zeros((ROWS_PS, DEGW), jnp.float32)
    ones = jnp.ones((CHUNK, DEGW), jnp.float32)

    degp = _deg_count(dst, zd, ones)
    aggp0 = _seg_sum(x, src, dst, zf)
    h1 = _layer_hidden(x, aggp0, degp, W_self0, W_neigh0,
                       b0.reshape(1, H), W_fc, b_fc.reshape(1, C))
    aggp1 = _seg_sum(h1, src, dst, zf)
    out = _layer_final(h1, aggp1, degp, W_self1, W_neigh1,
                       b1.reshape(1, H), W_fc, b_fc.reshape(1, C))
    return out


# speedup vs baseline: 1.1938x; 1.1938x over previous
import functools

import jax
import jax.numpy as jnp
from jax import lax
from jax.experimental import pallas as pl
from jax.experimental.pallas import tpu as pltpu
from jax.experimental.pallas import tpu_sc as plsc

N = 10000
E = 320000
D = 128
H = 128
C = 40

NC = 2
NS = 16
NW = NC * NS
EPW = E // NW
CHUNK = 125
NCHUNK = EPW // CHUNK
HC = NCHUNK // 2
ROWS_PS = 640
NPAD = NS * ROWS_PS
DEGW = 128

_mesh = plsc.VectorSubcoreMesh(core_axis_name="c", subcore_axis_name="s")


def _seg_sum_body(h_hbm, src_hbm, dst_hbm, zf_hbm, agg_hbm, agg_sh,
                  src_idx, dst_idx, rows0, rows1, sem0, sem1):
    c = lax.axis_index("c")
    s = lax.axis_index("s")
    wid = c * NS + s

    r0 = s * ROWS_PS
    pltpu.sync_copy(zf_hbm, agg_sh.at[pl.ds(r0, ROWS_PS)])
    plsc.subcore_barrier()

    def gather(i, buf, sem):
        pltpu.async_copy(h_hbm.at[src_idx.at[i]], buf, sem)

    def gather_wait(i, buf, sem):
        pltpu.make_async_copy(h_hbm.at[src_idx.at[i]], buf, sem).wait()

    def scat(i, buf):
        pltpu.sync_copy(buf, agg_sh.at[dst_idx.at[i]], add=True)

    for h in range(2):
        pltpu.sync_copy(src_hbm.at[wid, pl.ds(h * HC, HC)], src_idx)
        pltpu.sync_copy(dst_hbm.at[wid, pl.ds(h * HC, HC)], dst_idx)
        gather(0, rows0, sem0)

        @pl.loop(0, HC // 2 - 1)
        def _(j):
            i = 2 * j
            gather(i + 1, rows1, sem1)
            gather_wait(i, rows0, sem0)
            scat(i, rows0)
            gather(i + 2, rows0, sem0)
            gather_wait(i + 1, rows1, sem1)
            scat(i + 1, rows1)

        gather(HC - 1, rows1, sem1)
        gather_wait(HC - 2, rows0, sem0)
        scat(HC - 2, rows0)
        gather_wait(HC - 1, rows1, sem1)
        scat(HC - 1, rows1)

    plsc.subcore_barrier()
    pltpu.sync_copy(agg_sh.at[pl.ds(r0, ROWS_PS)],
                    agg_hbm.at[c, pl.ds(r0, ROWS_PS)])


def _deg_body(dst_hbm, zd_hbm, ones_hbm, deg_hbm, deg_sh, dst_idx, ones_v):
    c = lax.axis_index("c")
    s = lax.axis_index("s")
    wid = c * NS + s
    r0 = s * ROWS_PS
    pltpu.sync_copy(zd_hbm, deg_sh.at[pl.ds(r0, ROWS_PS)])
    pltpu.sync_copy(ones_hbm, ones_v)
    pltpu.sync_copy(dst_hbm.at[wid], dst_idx)
    plsc.subcore_barrier()

    @pl.loop(0, NCHUNK)
    def _(i):
        pltpu.sync_copy(ones_v, deg_sh.at[dst_idx.at[i]], add=True)

    plsc.subcore_barrier()
    pltpu.sync_copy(deg_sh.at[pl.ds(r0, ROWS_PS)],
                    deg_hbm.at[c, pl.ds(r0, ROWS_PS)])


_seg_sum = pl.kernel(
    _seg_sum_body,
    out_type=jax.ShapeDtypeStruct((NC, NPAD, D), jnp.float32),
    mesh=_mesh,
    scratch_types=[
        pltpu.VMEM_SHARED((NPAD, D), jnp.float32),
        pltpu.VMEM((HC, CHUNK), jnp.int32),
        pltpu.VMEM((HC, CHUNK), jnp.int32),
        pltpu.VMEM((CHUNK, D), jnp.float32),
        pltpu.VMEM((CHUNK, D), jnp.float32),
        pltpu.SemaphoreType.DMA,
        pltpu.SemaphoreType.DMA,
    ],
)

_deg_count = pl.kernel(
    _deg_body,
    out_type=jax.ShapeDtypeStruct((NC, NPAD, DEGW), jnp.float32),
    mesh=_mesh,
    scratch_types=[
        pltpu.VMEM_SHARED((NPAD, DEGW), jnp.float32),
        pltpu.VMEM((NCHUNK, CHUNK), jnp.int32),
        pltpu.VMEM((CHUNK, DEGW), jnp.float32),
    ],
)


BN = 1000


def _layer_body(out_w, h_ref, aggp_ref, degp_ref, ws_ref, wn_ref, b_ref,
                wfc_ref, bfc_ref, out_ref):
    agg = aggp_ref[0] + aggp_ref[1]
    deg = degp_ref[0, :, 0] + degp_ref[1, :, 0]
    neigh = agg * (1.0 / jnp.maximum(deg, 1.0))[:, None]
    hh = (jnp.dot(h_ref[...], ws_ref[...], preferred_element_type=jnp.float32)
          + jnp.dot(neigh, wn_ref[...], preferred_element_type=jnp.float32)
          + b_ref[...])
    hh = jnp.maximum(hh, 0.0)
    if out_w is None:
        out_ref[...] = hh
    else:
        out_ref[...] = (jnp.dot(hh, wfc_ref[...],
                                preferred_element_type=jnp.float32)
                        + bfc_ref[...])


def _make_layer(out_w):
    specs = [
        pl.BlockSpec((BN, D), lambda i: (i, 0)),
        pl.BlockSpec((NC, BN, D), lambda i: (0, i, 0)),
        pl.BlockSpec((NC, BN, DEGW), lambda i: (0, i, 0)),
        pl.BlockSpec((D, H), lambda i: (0, 0)),
        pl.BlockSpec((D, H), lambda i: (0, 0)),
        pl.BlockSpec((1, H), lambda i: (0, 0)),
        pl.BlockSpec((H, C), lambda i: (0, 0)),
        pl.BlockSpec((1, C), lambda i: (0, 0)),
    ]
    width = H if out_w is None else out_w
    return pl.pallas_call(
        functools.partial(_layer_body, out_w),
        grid=(N // BN,),
        in_specs=specs,
        out_specs=pl.BlockSpec((BN, width), lambda i: (i, 0)),
        out_shape=jax.ShapeDtypeStruct((N, width), jnp.float32),
    )


_layer_hidden = _make_layer(None)
_layer_final = _make_layer(C)


def kernel(x, edge_index, W_self0, W_neigh0, b0, W_self1, W_neigh1, b1,
           W_fc, b_fc):
    src = edge_index[0].reshape(NW, NCHUNK, CHUNK)
    dst = edge_index[1].reshape(NW, NCHUNK, CHUNK)
    zf = jnp.zeros((ROWS_PS, D), jnp.float32)
    zd = jnp.zeros((ROWS_PS, DEGW), jnp.float32)
    ones = jnp.ones((CHUNK, DEGW), jnp.float32)

    degp = _deg_count(dst, zd, ones)
    aggp0 = _seg_sum(x, src, dst, zf)
    h1 = _layer_hidden(x, aggp0, degp, W_self0, W_neigh0,
                       b0.reshape(1, H), W_fc, b_fc.reshape(1, C))
    aggp1 = _seg_sum(h1, src, dst, zf)
    out = _layer_final(h1, aggp1, degp, W_self1, W_neigh1,
                       b1.reshape(1, H), W_fc, b_fc.reshape(1, C))
    return out
